# Initial kernel scaffold; baseline (speedup 1.0000x reference)
import jax
import jax.numpy as jnp
from jax.experimental import pallas as pl


def kernel(user_emb, item_emb, edge_vals, edge_index):
    raise NotImplementedError("write your pallas kernel here")


# speedup vs baseline: 7.4994x; 7.4994x over previous
import jax
import jax.numpy as jnp
from jax import lax
from jax.experimental import pallas as pl
from jax.experimental.pallas import tpu as pltpu, tpu_sc as plsc

N_USERS = 50000
N_ITEMS = 50000
EMB = 32
N_LAYERS = 3
N = N_USERS + N_ITEMS
E = 1600000

NC = 2
NS = 16
HEMB = EMB // NC
CHUNK = 128
BLK = 8
ROWS_PER_TILE = 784
ROWS = ROWS_PER_TILE * NS
E_PAD = ROWS * CHUNK
HALF = N // NC
HALF_PAD = 50048
N_PAD = NC * HALF_PAD
OUT_STRIPE = N_PAD // NS
OUT_CHUNK = 272


def _body(emb_hbm, col_hbm, row_hbm, val_hbm, work_hbm, accm_hbm,
          acc_sh, colb, rowb, valb, cidx, grows, zb, nbuf, abuf, sem):
    c = lax.axis_index("c")
    s = lax.axis_index("s")
    coff = c * N_PAD
    row0 = s * ROWS_PER_TILE
    out0 = s * OUT_STRIPE

    zero16 = jnp.zeros((16,), jnp.float32)

    def _z(i, _):
        zb[i, pl.ds(0, 16)] = zero16
        return 0

    lax.fori_loop(0, OUT_CHUNK, _z, 0)
    for q in range(OUT_STRIPE // OUT_CHUNK):
        pltpu.sync_copy(zb, acc_sh.at[pl.ds(out0 + q * OUT_CHUNK, OUT_CHUNK)])
    plsc.subcore_barrier()

    for L in range(N_LAYERS):
        gsrc = emb_hbm if L == 0 else work_hbm

        def _outer(b, _):
            rbase = row0 + b * BLK
            pltpu.sync_copy(col_hbm.at[pl.ds(rbase, BLK)], colb)
            pltpu.sync_copy(row_hbm.at[pl.ds(rbase, BLK)], rowb)
            pltpu.sync_copy(val_hbm.at[pl.ds(rbase, BLK)], valb)
            for j in range(BLK):
                def _prep(k, _):
                    cidx[pl.ds(k * 16, 16)] = colb[j, pl.ds(k * 16, 16)] + coff
                    return 0

                lax.fori_loop(0, CHUNK // 16, _prep, 0)

                pltpu.async_copy(gsrc.at[cidx], grows, sem).wait()

                def _mul(g, _):
                    mv = valb[j, pl.ds(g * 16, 16)]
                    base = g * 16
                    for i in range(16):
                        grows[base + i, pl.ds(0, 16)] = (
                            grows[base + i, pl.ds(0, 16)] * mv[i])
                    return 0

                lax.fori_loop(0, CHUNK // 16, _mul, 0)

                pltpu.sync_copy(grows, acc_sh.at[rowb.at[j]], add=True)
            return 0

        lax.fori_loop(0, ROWS_PER_TILE // BLK, _outer, 0)
        plsc.subcore_barrier()

        msrc = emb_hbm if L == 0 else accm_hbm
        last = L == N_LAYERS - 1
        for q in range(OUT_STRIPE // OUT_CHUNK):
            r0 = out0 + q * OUT_CHUNK
            pltpu.sync_copy(acc_sh.at[pl.ds(r0, OUT_CHUNK)], nbuf)
            if not last:
                pltpu.sync_copy(zb, acc_sh.at[pl.ds(r0, OUT_CHUNK)])
                pltpu.sync_copy(nbuf, work_hbm.at[pl.ds(coff + r0, OUT_CHUNK)])
            pltpu.sync_copy(msrc.at[pl.ds(coff + r0, OUT_CHUNK)], abuf)

            def _acc(i, _):
                a = abuf[i, pl.ds(0, 16)] + nbuf[i, pl.ds(0, 16)]
                if last:
                    a = a * jnp.float32(1.0 / (N_LAYERS + 1))
                abuf[i, pl.ds(0, 16)] = a
                return 0

            lax.fori_loop(0, OUT_CHUNK, _acc, 0)
            pltpu.sync_copy(abuf, accm_hbm.at[pl.ds(coff + r0, OUT_CHUNK)])
        plsc.subcore_barrier()


def _propagate(emb_flat, col2d, row2d, val2d):
    mesh = plsc.VectorSubcoreMesh(core_axis_name="c", subcore_axis_name="s",
                                  num_cores=NC, num_subcores=NS)
    return pl.kernel(
        _body,
        out_type=(
            jax.ShapeDtypeStruct((NC * N_PAD, HEMB), jnp.float32),
            jax.ShapeDtypeStruct((NC * N_PAD, HEMB), jnp.float32),
        ),
        mesh=mesh,
        scratch_types=[
            pltpu.VMEM_SHARED((N_PAD, HEMB), jnp.float32),
            pltpu.VMEM((BLK, CHUNK), jnp.int32),
            pltpu.VMEM((BLK, CHUNK), jnp.int32),
            pltpu.VMEM((BLK, CHUNK), jnp.float32),
            pltpu.VMEM((CHUNK,), jnp.int32),
            pltpu.VMEM((CHUNK, HEMB), jnp.float32),
            pltpu.VMEM((OUT_CHUNK, HEMB), jnp.float32),
            pltpu.VMEM((OUT_CHUNK, HEMB), jnp.float32),
            pltpu.VMEM((OUT_CHUNK, HEMB), jnp.float32),
            pltpu.SemaphoreType.DMA,
        ],
        compiler_params=pltpu.CompilerParams(use_tc_tiling_on_sc=False),
    )(emb_flat, col2d, row2d, val2d)


def kernel(user_emb, item_emb, edge_vals, edge_index):
    padrows = jnp.zeros((HALF_PAD - HALF, EMB), jnp.float32)
    emb = jnp.concatenate([user_emb, padrows, item_emb, padrows], axis=0)
    emb_flat = (emb.reshape(N_PAD, NC, HEMB)
                .transpose(1, 0, 2)
                .reshape(NC * N_PAD, HEMB))
    row = edge_index[0]
    col = edge_index[1]
    shift = jnp.int32(HALF_PAD - HALF)
    col = col + jnp.where(col >= HALF, shift, 0).astype(jnp.int32)
    row = row + jnp.where(row >= HALF, shift, 0).astype(jnp.int32)
    pad = E_PAD - E
    col2d = jnp.pad(col, (0, pad)).reshape(ROWS, CHUNK)
    row2d = jnp.pad(row, (0, pad)).reshape(ROWS, CHUNK)
    val2d = jnp.pad(edge_vals, (0, pad)).reshape(ROWS, CHUNK)
    _, accm = _propagate(emb_flat, col2d, row2d, val2d)
    out = (accm.reshape(NC, N_PAD, HEMB)
           .transpose(1, 0, 2)
           .reshape(N_PAD, EMB))
    return out[:N_USERS], out[HALF_PAD:HALF_PAD + N_ITEMS]


# speedup vs baseline: 8.7994x; 1.1733x over previous
import jax
import jax.numpy as jnp
from jax import lax
from jax.experimental import pallas as pl
from jax.experimental.pallas import tpu as pltpu, tpu_sc as plsc

N_USERS = 50000
N_ITEMS = 50000
EMB = 32
N_LAYERS = 3
N = N_USERS + N_ITEMS
E = 1600000

NC = 2
NS = 16
HEMB = EMB // NC
CHUNK = 128
BLK = 8
ROWS_PER_TILE = 784
ROWS = ROWS_PER_TILE * NS
E_PAD = ROWS * CHUNK
HALF = N // NC
HALF_PAD = 50048
N_PAD = NC * HALF_PAD
OUT_STRIPE = N_PAD // NS
OUT_CHUNK = 272


def _body(emb_hbm, col_hbm, row_hbm, val_hbm, work_hbm, accm_hbm,
          acc_sh, colb, rowb, valb, cidx, grows, zb, nbuf, abuf, gsem, ssem):
    c = lax.axis_index("c")
    s = lax.axis_index("s")
    coff = c * N_PAD
    row0 = s * ROWS_PER_TILE
    out0 = s * OUT_STRIPE

    zero16 = jnp.zeros((16,), jnp.float32)

    def _z(i, _):
        zb[i, pl.ds(0, 16)] = zero16
        return 0

    lax.fori_loop(0, OUT_CHUNK, _z, 0)
    for q in range(OUT_STRIPE // OUT_CHUNK):
        pltpu.sync_copy(zb, acc_sh.at[pl.ds(out0 + q * OUT_CHUNK, OUT_CHUNK)])
    plsc.subcore_barrier()

    for L in range(N_LAYERS):
        gsrc = emb_hbm if L == 0 else work_hbm

        def _outer(b, _):
            rbase = row0 + b * BLK
            pltpu.sync_copy(col_hbm.at[pl.ds(rbase, BLK)], colb)
            pltpu.sync_copy(row_hbm.at[pl.ds(rbase, BLK)], rowb)
            pltpu.sync_copy(val_hbm.at[pl.ds(rbase, BLK)], valb)

            def _prep(j, p):
                def _k(k, _):
                    cidx[p, pl.ds(k * 16, 16)] = (
                        colb[j, pl.ds(k * 16, 16)] + coff)
                    return 0

                lax.fori_loop(0, CHUNK // 16, _k, 0)

            def _fire_gather(j, p):
                return pltpu.async_copy(gsrc.at[cidx.at[p]],
                                        grows.at[p], gsem[p])

            _prep(0, 0)
            gd = [None, None]
            sd = [None, None]
            gd[0] = _fire_gather(0, 0)
            for j in range(BLK):
                p = j % 2
                q = 1 - p
                gd[p].wait()
                if j + 1 < BLK:
                    _prep(j + 1, q)
                    if sd[q] is not None:
                        sd[q].wait()
                        sd[q] = None
                    gd[q] = _fire_gather(j + 1, q)

                def _mul(g, _):
                    mv = valb[j, pl.ds(g * 16, 16)]
                    base = g * 16
                    for i in range(16):
                        grows[p, base + i, pl.ds(0, 16)] = (
                            grows[p, base + i, pl.ds(0, 16)] * mv[i])
                    return 0

                lax.fori_loop(0, CHUNK // 16, _mul, 0)

                sd[p] = pltpu.async_copy(grows.at[p],
                                         acc_sh.at[rowb.at[j]],
                                         ssem[p], add=True)
            for p in range(2):
                if sd[p] is not None:
                    sd[p].wait()
            return 0

        lax.fori_loop(0, ROWS_PER_TILE // BLK, _outer, 0)
        plsc.subcore_barrier()

        msrc = emb_hbm if L == 0 else accm_hbm
        last = L == N_LAYERS - 1
        for q in range(OUT_STRIPE // OUT_CHUNK):
            r0 = out0 + q * OUT_CHUNK
            pltpu.sync_copy(acc_sh.at[pl.ds(r0, OUT_CHUNK)], nbuf)
            if not last:
                pltpu.sync_copy(zb, acc_sh.at[pl.ds(r0, OUT_CHUNK)])
                pltpu.sync_copy(nbuf, work_hbm.at[pl.ds(coff + r0, OUT_CHUNK)])
            pltpu.sync_copy(msrc.at[pl.ds(coff + r0, OUT_CHUNK)], abuf)

            def _acc(i, _):
                a = abuf[i, pl.ds(0, 16)] + nbuf[i, pl.ds(0, 16)]
                if last:
                    a = a * jnp.float32(1.0 / (N_LAYERS + 1))
                abuf[i, pl.ds(0, 16)] = a
                return 0

            lax.fori_loop(0, OUT_CHUNK, _acc, 0)
            pltpu.sync_copy(abuf, accm_hbm.at[pl.ds(coff + r0, OUT_CHUNK)])
        plsc.subcore_barrier()


def _propagate(emb_flat, col2d, row2d, val2d):
    mesh = plsc.VectorSubcoreMesh(core_axis_name="c", subcore_axis_name="s",
                                  num_cores=NC, num_subcores=NS)
    return pl.kernel(
        _body,
        out_type=(
            jax.ShapeDtypeStruct((NC * N_PAD, HEMB), jnp.float32),
            jax.ShapeDtypeStruct((NC * N_PAD, HEMB), jnp.float32),
        ),
        mesh=mesh,
        scratch_types=[
            pltpu.VMEM_SHARED((N_PAD, HEMB), jnp.float32),
            pltpu.VMEM((BLK, CHUNK), jnp.int32),
            pltpu.VMEM((BLK, CHUNK), jnp.int32),
            pltpu.VMEM((BLK, CHUNK), jnp.float32),
            pltpu.VMEM((2, CHUNK), jnp.int32),
            pltpu.VMEM((2, CHUNK, HEMB), jnp.float32),
            pltpu.VMEM((OUT_CHUNK, HEMB), jnp.float32),
            pltpu.VMEM((OUT_CHUNK, HEMB), jnp.float32),
            pltpu.VMEM((OUT_CHUNK, HEMB), jnp.float32),
            (pltpu.SemaphoreType.DMA, pltpu.SemaphoreType.DMA),
            (pltpu.SemaphoreType.DMA, pltpu.SemaphoreType.DMA),
        ],
        compiler_params=pltpu.CompilerParams(use_tc_tiling_on_sc=False),
    )(emb_flat, col2d, row2d, val2d)


def kernel(user_emb, item_emb, edge_vals, edge_index):
    padrows = jnp.zeros((HALF_PAD - HALF, EMB), jnp.float32)
    emb = jnp.concatenate([user_emb, padrows, item_emb, padrows], axis=0)
    emb_flat = (emb.reshape(N_PAD, NC, HEMB)
                .transpose(1, 0, 2)
                .reshape(NC * N_PAD, HEMB))
    row = edge_index[0]
    col = edge_index[1]
    shift = jnp.int32(HALF_PAD - HALF)
    col = col + jnp.where(col >= HALF, shift, 0).astype(jnp.int32)
    row = row + jnp.where(row >= HALF, shift, 0).astype(jnp.int32)
    pad = E_PAD - E
    col2d = jnp.pad(col, (0, pad)).reshape(ROWS, CHUNK)
    row2d = jnp.pad(row, (0, pad)).reshape(ROWS, CHUNK)
    val2d = jnp.pad(edge_vals, (0, pad)).reshape(ROWS, CHUNK)
    _, accm = _propagate(emb_flat, col2d, row2d, val2d)
    out = (accm.reshape(NC, N_PAD, HEMB)
           .transpose(1, 0, 2)
           .reshape(N_PAD, EMB))
    return out[:N_USERS], out[HALF_PAD:HALF_PAD + N_ITEMS]


# speedup vs baseline: 14.8701x; 1.6899x over previous
import jax
import jax.numpy as jnp
from jax import lax
from jax.experimental import pallas as pl
from jax.experimental.pallas import tpu as pltpu, tpu_sc as plsc

N_USERS = 50000
N_ITEMS = 50000
EMB = 32
N_LAYERS = 3
N = N_USERS + N_ITEMS
E = 1600000

NC = 2
NS = 16
HEMB = EMB // NC
CHUNK = 128
BLK = 8
RING = 4
ROWS_PER_TILE = 784
ROWS = ROWS_PER_TILE * NS
E_PAD = ROWS * CHUNK
HALF = N // NC
HALF_PAD = 50048
N_PAD = NC * HALF_PAD
OUT_STRIPE = N_PAD // NS
OUT_CHUNK = 272


def _body(emb_hbm, edge_hbm, work_hbm, accm_hbm,
          acc_sh, edgb, cidx, grows, zb, nbuf, abuf, gsem, ssem):
    c = lax.axis_index("c")
    s = lax.axis_index("s")
    coff = c * N_PAD
    row0 = s * ROWS_PER_TILE
    out0 = s * OUT_STRIPE

    zero16 = jnp.zeros((16,), jnp.float32)

    def _z(i, _):
        zb[i, pl.ds(0, 16)] = zero16
        return 0

    lax.fori_loop(0, OUT_CHUNK, _z, 0)
    for q in range(OUT_STRIPE // OUT_CHUNK):
        pltpu.sync_copy(zb, acc_sh.at[pl.ds(out0 + q * OUT_CHUNK, OUT_CHUNK)])
    plsc.subcore_barrier()

    for L in range(N_LAYERS):
        gsrc = emb_hbm if L == 0 else work_hbm

        def _outer(b, _):
            rbase = row0 + b * BLK
            pltpu.sync_copy(edge_hbm.at[pl.ds(rbase, BLK)], edgb)

            def _prep(j, p):
                def _k(k, _):
                    cidx[p, pl.ds(k * 16, 16)] = (
                        edgb[j, 0, pl.ds(k * 16, 16)] + coff)
                    return 0

                lax.fori_loop(0, CHUNK // 16, _k, 0)

            def _fire_gather(j, p):
                return pltpu.async_copy(gsrc.at[cidx.at[p]],
                                        grows.at[p], gsem[p])

            gd = [None] * RING
            sd = [None] * RING
            for j in range(RING - 1):
                _prep(j, j)
                gd[j] = _fire_gather(j, j)
            for j in range(BLK):
                p = j % RING
                gd[p].wait()

                def _mul(g, _):
                    mv = plsc.bitcast(edgb[j, 2, pl.ds(g * 16, 16)],
                                      jnp.float32)
                    base = g * 16
                    for i in range(16):
                        grows[p, base + i, pl.ds(0, 16)] = (
                            grows[p, base + i, pl.ds(0, 16)] * mv[i])
                    return 0

                lax.fori_loop(0, CHUNK // 16, _mul, 0)

                sd[p] = pltpu.async_copy(grows.at[p],
                                         acc_sh.at[edgb.at[j, 1]],
                                         ssem[p], add=True)

                nj = j + RING - 1
                if nj < BLK:
                    np_ = nj % RING
                    _prep(nj, np_)
                    if sd[np_] is not None:
                        sd[np_].wait()
                        sd[np_] = None
                    gd[np_] = _fire_gather(nj, np_)
            for p in range(RING):
                if sd[p] is not None:
                    sd[p].wait()
            return 0

        lax.fori_loop(0, ROWS_PER_TILE // BLK, _outer, 0)
        plsc.subcore_barrier()

        msrc = emb_hbm if L == 0 else accm_hbm
        last = L == N_LAYERS - 1
        for q in range(OUT_STRIPE // OUT_CHUNK):
            r0 = out0 + q * OUT_CHUNK
            pltpu.sync_copy(acc_sh.at[pl.ds(r0, OUT_CHUNK)], nbuf)
            if not last:
                pltpu.sync_copy(zb, acc_sh.at[pl.ds(r0, OUT_CHUNK)])
                pltpu.sync_copy(nbuf, work_hbm.at[pl.ds(coff + r0, OUT_CHUNK)])
            pltpu.sync_copy(msrc.at[pl.ds(coff + r0, OUT_CHUNK)], abuf)

            def _acc(i, _):
                a = abuf[i, pl.ds(0, 16)] + nbuf[i, pl.ds(0, 16)]
                if last:
                    a = a * jnp.float32(1.0 / (N_LAYERS + 1))
                abuf[i, pl.ds(0, 16)] = a
                return 0

            lax.fori_loop(0, OUT_CHUNK, _acc, 0)
            pltpu.sync_copy(abuf, accm_hbm.at[pl.ds(coff + r0, OUT_CHUNK)])
        plsc.subcore_barrier()


def _propagate(emb_flat, edges):
    mesh = plsc.VectorSubcoreMesh(core_axis_name="c", subcore_axis_name="s",
                                  num_cores=NC, num_subcores=NS)
    return pl.kernel(
        _body,
        out_type=(
            jax.ShapeDtypeStruct((NC * N_PAD, HEMB), jnp.float32),
            jax.ShapeDtypeStruct((NC * N_PAD, HEMB), jnp.float32),
        ),
        mesh=mesh,
        scratch_types=[
            pltpu.VMEM_SHARED((N_PAD, HEMB), jnp.float32),
            pltpu.VMEM((BLK, 3, CHUNK), jnp.int32),
            pltpu.VMEM((RING, CHUNK), jnp.int32),
            pltpu.VMEM((RING, CHUNK, HEMB), jnp.float32),
            pltpu.VMEM((OUT_CHUNK, HEMB), jnp.float32),
            pltpu.VMEM((OUT_CHUNK, HEMB), jnp.float32),
            pltpu.VMEM((OUT_CHUNK, HEMB), jnp.float32),
            tuple(pltpu.SemaphoreType.DMA for _ in range(RING)),
            tuple(pltpu.SemaphoreType.DMA for _ in range(RING)),
        ],
        compiler_params=pltpu.CompilerParams(use_tc_tiling_on_sc=False,
                                             needs_layout_passes=False),
    )(emb_flat, edges)


def kernel(user_emb, item_emb, edge_vals, edge_index):
    padrows = jnp.zeros((HALF_PAD - HALF, EMB), jnp.float32)
    emb = jnp.concatenate([user_emb, padrows, item_emb, padrows], axis=0)
    emb_flat = (emb.reshape(N_PAD, NC, HEMB)
                .transpose(1, 0, 2)
                .reshape(NC * N_PAD, HEMB))
    row = edge_index[0]
    col = edge_index[1]
    shift = jnp.int32(HALF_PAD - HALF)
    col = col + jnp.where(col >= HALF, shift, 0).astype(jnp.int32)
    row = row + jnp.where(row >= HALF, shift, 0).astype(jnp.int32)
    pad = E_PAD - E
    col2d = jnp.pad(col, (0, pad)).reshape(ROWS, 1, CHUNK)
    row2d = jnp.pad(row, (0, pad)).reshape(ROWS, 1, CHUNK)
    val2d = lax.bitcast_convert_type(
        jnp.pad(edge_vals, (0, pad)).reshape(ROWS, 1, CHUNK), jnp.int32)
    edges = jnp.concatenate([col2d, row2d, val2d], axis=1)
    _, accm = _propagate(emb_flat, edges)
    out = (accm.reshape(NC, N_PAD, HEMB)
           .transpose(1, 0, 2)
           .reshape(N_PAD, EMB))
    return out[:N_USERS], out[HALF_PAD:HALF_PAD + N_ITEMS]
